# baseline (speedup 1.0000x reference)
import jax
import jax.numpy as jnp
from jax.experimental import pallas as pl


def _copy_body(x_ref, o_ref):
    o_ref[...] = x_ref[...]


def kernel(x):
    rows, cols = x.shape
    block_rows = 16384
    grid = (rows // block_rows,)
    return pl.pallas_call(
        _copy_body,
        out_shape=jax.ShapeDtypeStruct(x.shape, x.dtype),
        grid=grid,
        in_specs=[pl.BlockSpec((block_rows, cols), lambda i: (i, 0))],
        out_specs=pl.BlockSpec((block_rows, cols), lambda i: (i, 0)),
    )(x)


# speedup vs baseline: 1.0602x; 1.0602x over previous
import jax
import jax.numpy as jnp
from jax.experimental import pallas as pl
from jax.experimental.pallas import tpu as pltpu

_K = 4


def _ring_body(x_hbm, o_hbm, buf, sem_in, sem_out):
    rows = x_hbm.shape[0]
    c = rows // _K

    def in_cp(i):
        return pltpu.make_async_copy(
            x_hbm.at[pl.ds(i * c, c), :], buf.at[i % 2], sem_in)

    def out_cp(i):
        return pltpu.make_async_copy(
            buf.at[i % 2], o_hbm.at[pl.ds(i * c, c), :], sem_out)

    in_cp(0).start()
    in_cp(1).start()
    in_cp(0).wait()
    out_cp(0).start()
    in_cp(1).wait()
    out_cp(1).start()
    out_cp(0).wait()
    in_cp(2).start()
    out_cp(1).wait()
    in_cp(3).start()
    in_cp(2).wait()
    out_cp(2).start()
    in_cp(3).wait()
    out_cp(3).start()
    out_cp(2).wait()
    out_cp(3).wait()


def kernel(x):
    rows, cols = x.shape
    return pl.pallas_call(
        _ring_body,
        out_shape=jax.ShapeDtypeStruct(x.shape, x.dtype),
        in_specs=[pl.BlockSpec(memory_space=pl.ANY)],
        out_specs=pl.BlockSpec(memory_space=pl.ANY),
        scratch_shapes=[
            pltpu.VMEM((2, rows // _K, cols), x.dtype),
            pltpu.SemaphoreType.DMA,
            pltpu.SemaphoreType.DMA,
        ],
    )(x)
